# Initial kernel scaffold; baseline (speedup 1.0000x reference)
import jax
import jax.numpy as jnp
from jax.experimental import pallas as pl


def kernel(x, mf_indices):
    raise NotImplementedError("write your pallas kernel here")


# speedup vs baseline: 6901.2046x; 6901.2046x over previous
import jax
import jax.numpy as jnp
from jax.experimental import pallas as pl

_RULE_BLK = 512


def _antecedent_block(x_ref, idx_ref, o_ref):
    acc = None
    for i in range(8):
        idx = idx_ref[i : i + 1, :]
        x0 = x_ref[:, 3 * i : 3 * i + 1]
        x1 = x_ref[:, 3 * i + 1 : 3 * i + 2]
        x2 = x_ref[:, 3 * i + 2 : 3 * i + 3]
        v = jnp.where(idx == 0, x0, jnp.where(idx == 1, x1, x2))
        acc = v if acc is None else acc * v
    o_ref[:, :] = acc


def kernel(x, mf_indices):
    B, n_in, n_mfs = x.shape
    n_rules = mf_indices.shape[0]
    xf = x.reshape(B, n_in * n_mfs)
    idxT = mf_indices.astype(jnp.int32).T
    grid = (pl.cdiv(n_rules, _RULE_BLK),)
    return pl.pallas_call(
        _antecedent_block,
        grid=grid,
        in_specs=[
            pl.BlockSpec((B, n_in * n_mfs), lambda j: (0, 0)),
            pl.BlockSpec((n_in, _RULE_BLK), lambda j: (0, j)),
        ],
        out_specs=pl.BlockSpec((B, _RULE_BLK), lambda j: (0, j)),
        out_shape=jax.ShapeDtypeStruct((B, n_rules), jnp.float32),
    )(xf, idxT)
